# baseline (speedup 1.0000x reference)
import jax
import jax.numpy as jnp
from jax.experimental import pallas as pl

_B, _T, _F = 32, 2048, 768
_ROWS = _B * _T
_RBLK = 1024


def _mask_kernel(mask_ref, emb_ref, x_ref, out_ref):
    out_ref[...] = jnp.where(mask_ref[...], emb_ref[...], x_ref[...])


def kernel(x, mask, mask_emb):
    xr = x.reshape(_ROWS, _F)
    mr = mask.reshape(_ROWS, 1)
    emb = mask_emb.reshape(1, _F)
    out = pl.pallas_call(
        _mask_kernel,
        grid=(_ROWS // _RBLK,),
        in_specs=[
            pl.BlockSpec((_RBLK, 1), lambda i: (i, 0)),
            pl.BlockSpec((1, _F), lambda i: (0, 0)),
            pl.BlockSpec((_RBLK, _F), lambda i: (i, 0)),
        ],
        out_specs=pl.BlockSpec((_RBLK, _F), lambda i: (i, 0)),
        out_shape=jax.ShapeDtypeStruct((_ROWS, _F), x.dtype),
    )(mr, emb, xr)
    return out.reshape(_B, _T, _F)


# speedup vs baseline: 1.0123x; 1.0123x over previous
import jax
import jax.numpy as jnp
from jax.experimental import pallas as pl

_B, _T, _F = 32, 2048, 768
_ROWS = _B * _T
_RBLK = 4096


def _mask_kernel(mask_ref, emb_ref, x_ref, out_ref):
    out_ref[...] = jnp.where(mask_ref[...], emb_ref[...], x_ref[...])


def kernel(x, mask, mask_emb):
    xr = x.reshape(_ROWS, _F)
    mr = mask.reshape(_ROWS, 1)
    emb = mask_emb.reshape(1, _F)
    out = pl.pallas_call(
        _mask_kernel,
        grid=(_ROWS // _RBLK,),
        in_specs=[
            pl.BlockSpec((_RBLK, 1), lambda i: (i, 0)),
            pl.BlockSpec((1, _F), lambda i: (0, 0)),
            pl.BlockSpec((_RBLK, _F), lambda i: (i, 0)),
        ],
        out_specs=pl.BlockSpec((_RBLK, _F), lambda i: (i, 0)),
        out_shape=jax.ShapeDtypeStruct((_ROWS, _F), x.dtype),
    )(mr, emb, xr)
    return out.reshape(_B, _T, _F)
